# Initial kernel scaffold; baseline (speedup 1.0000x reference)
import jax
import jax.numpy as jnp
from jax.experimental import pallas as pl


def kernel(x, edge_index, edge_weight, batch, edge_index_neighbor, edge_weight_neighbor, batch_neighbor, Wo0, bo0, Wn0, bn0, Wf0, bf0, Wo1, bo1, Wn1, bn1, Wf1, bf1):
    raise NotImplementedError("write your pallas kernel here")


# speedup vs baseline: 4.5821x; 4.5821x over previous
import functools

import jax
import jax.numpy as jnp
from jax import lax
from jax.experimental import pallas as pl
from jax.experimental.pallas import tpu as pltpu
from jax.experimental.pallas import tpu_sc as plsc

N = 10000
E = 320000
D = 128
HD = 64
G = 128
NS = 16
EPT = E // NS
CH = 2000
ROWS = E // 128
SEG = N // NS

_F32 = jnp.float32
_I32 = jnp.int32
_HIGH = lax.Precision.HIGHEST


def _dot(a, b):
    return lax.dot_general(a, b, (((1,), (0,)), ((), ())),
                           precision=_HIGH, preferred_element_type=_F32)


def _deg_body(dstf, ewf, zN, degp, deg_v, dbuf, wbuf):
    c = lax.axis_index("c")
    t = lax.axis_index("s")
    pltpu.sync_copy(zN, deg_v)
    e0 = c * E + t * EPT

    def chunk(k, _):
        base = e0 + k * CH
        pltpu.sync_copy(dstf.at[pl.ds(base, CH)], dbuf)
        pltpu.sync_copy(ewf.at[pl.ds(base, CH)], wbuf)

        def inner(i, _):
            idx = dbuf[pl.ds(i * 16, 16)]
            w = wbuf[pl.ds(i * 16, 16)]
            plsc.addupdate_scatter(deg_v, [idx], w)
            return 0

        lax.fori_loop(0, CH // 16, inner, 0)
        return 0

    lax.fori_loop(0, EPT // CH, chunk, 0)
    pltpu.sync_copy(deg_v, degp.at[pl.ds(pl.multiple_of((c * NS + t) * N, 8), N)])


def _agg_body(ho2, hn2, srco, dsto, ewo, srcn, dstn, ewn, z2d, So, Sn,
              So_sp, Sn_sp, src_v, adj_v, dst_v, ew_v, rows_v):
    c = lax.axis_index("c")
    t = lax.axis_index("s")
    cbase = pl.multiple_of(c * N, 8)

    @pl.when(t == 0)
    def _():
        pltpu.sync_copy(z2d, So_sp)
        pltpu.sync_copy(z2d, Sn_sp)

    plsc.subcore_barrier()

    r0 = t * 156 + jnp.minimum(t, 4)
    nrows = 156 + (t < 4).astype(_I32)

    for (src, dst, ew, tab, S) in ((srco, dsto, ewo, ho2, So_sp),
                                   (srcn, dstn, ewn, hn2, Sn_sp)):
        def row_body(r, _, src=src, dst=dst, ew=ew, tab=tab, S=S):
            pltpu.sync_copy(src.at[pl.ds(r * 128, 128)], src_v)
            pltpu.sync_copy(dst.at[pl.ds(r * 128, 128)], dst_v)
            pltpu.sync_copy(ew.at[pl.ds(r * 128, 128)], ew_v)
            for j in range(8):
                adj_v[pl.ds(j * 16, 16)] = src_v[pl.ds(j * 16, 16)] + cbase
            pltpu.sync_copy(tab.at[adj_v], rows_v)

            def scale(e, _):
                er = jnp.full((16,), e, _I32)
                bw = plsc.load_gather(ew_v, [er])
                for f in range(4):
                    col = jnp.arange(16, dtype=_I32) + f * 16
                    v = plsc.load_gather(rows_v, [er, col])
                    plsc.store_scatter(rows_v, [er, col], v * bw)
                return 0

            lax.fori_loop(0, 128, scale, 0)
            pltpu.sync_copy(rows_v, S.at[dst_v], add=True)
            return 0

        lax.fori_loop(r0, r0 + nrows, row_body, 0)

    plsc.subcore_barrier()

    @pl.when(t == 0)
    def _():
        pltpu.sync_copy(So_sp, So.at[pl.ds(cbase, N)])
        pltpu.sync_copy(Sn_sp, Sn.at[pl.ds(cbase, N)])


def _sc_mesh():
    return plsc.VectorSubcoreMesh(core_axis_name="c", subcore_axis_name="s")


def _run_deg(dstf, ewf, zN):
    return pl.kernel(
        _deg_body,
        out_type=jax.ShapeDtypeStruct((2 * NS * N,), _F32),
        mesh=_sc_mesh(),
        scratch_types=[
            pltpu.VMEM((N,), _F32),
            pltpu.VMEM((CH,), _I32),
            pltpu.VMEM((CH,), _F32),
        ],
        compiler_params=pltpu.CompilerParams(needs_layout_passes=False, use_tc_tiling_on_sc=False),
        name="gcn_deg",
    )(dstf, ewf, zN)


def _run_agg(ho2, hn2, srco, dsto, ewo, srcn, dstn, ewn, z2d):
    return pl.kernel(
        _agg_body,
        out_type=(jax.ShapeDtypeStruct((2 * N, HD), _F32),
                  jax.ShapeDtypeStruct((2 * N, HD), _F32)),
        mesh=_sc_mesh(),
        scratch_types=[
            pltpu.VMEM_SHARED((N, HD), _F32),
            pltpu.VMEM_SHARED((N, HD), _F32),
            pltpu.VMEM((128,), _I32),
            pltpu.VMEM((128,), _I32),
            pltpu.VMEM((128,), _I32),
            pltpu.VMEM((128,), _F32),
            pltpu.VMEM((128, HD), _F32),
        ],
        compiler_params=pltpu.CompilerParams(needs_layout_passes=False, use_tc_tiling_on_sc=False),
        name="gcn_agg",
    )(ho2, hn2, srco, dsto, ewo, srcn, dstn, ewn, z2d)


def _prepw_body(wo0, wn0, wf0, bo0, bn0, bf0, wo1, wn1, wf1, bo1, bn1, bf1,
                wco0, wcn0, bc0, wco1, wcn1, bc1):
    for (wo, wn, wf, bo, bn, bf, wco, wcn, bc) in (
            (wo0, wn0, wf0, bo0, bn0, bf0, wco0, wcn0, bc0),
            (wo1, wn1, wf1, bo1, bn1, bf1, wco1, wcn1, bc1)):
        wfv = wf[...]
        wft = wfv[:D, :]
        wfb = wfv[D:, :]
        wco[...] = _dot(wo[...], wft)
        wcn[...] = _dot(wn[...], wfb)
        bc[...] = _dot(bo[...], wft) + _dot(bn[...], wfb) + bf[...]


def _prepd_body(degp_ref, dinv_ref):
    s = jnp.sum(degp_ref[...], axis=(0, 1))
    dinv_ref[...] = lax.rsqrt(1.0 + s)[None]


NB = 400
NBLK = N // NB


def _mm_body(x_ref, wo_ref, wn_ref, dvo_ref, dvn_ref, ho_ref, hn_ref):
    xb = x_ref[...]
    ho = _dot(xb, wo_ref[...]) * dvo_ref[...]
    hn = _dot(xb, wn_ref[...]) * dvn_ref[...]
    ho_ref[...] = jnp.stack([ho[:, :HD], ho[:, HD:]], axis=0)
    hn_ref[...] = jnp.stack([hn[:, :HD], hn[:, HD:]], axis=0)


def _run_mm(xin, wco, wcn, dvo, dvn):
    return pl.pallas_call(
        _mm_body,
        grid=(NBLK,),
        in_specs=[
            pl.BlockSpec((NB, D), lambda i: (i, 0)),
            pl.BlockSpec((D, D), lambda i: (0, 0)),
            pl.BlockSpec((D, D), lambda i: (0, 0)),
            pl.BlockSpec((NB, 1), lambda i: (i, 0)),
            pl.BlockSpec((NB, 1), lambda i: (i, 0)),
        ],
        out_specs=[
            pl.BlockSpec((2, NB, HD), lambda i: (0, i, 0)),
            pl.BlockSpec((2, NB, HD), lambda i: (0, i, 0)),
        ],
        out_shape=[jax.ShapeDtypeStruct((2, N, HD), _F32)] * 2,
    )(xin, wco, wcn, dvo, dvn)


def _comb_body(so_ref, sn_ref, ho_ref, hn_ref, dvo_ref, dvn_ref, bc_ref,
               out_ref):
    so = jnp.concatenate([so_ref[0], so_ref[1]], axis=1)
    sn = jnp.concatenate([sn_ref[0], sn_ref[1]], axis=1)
    ho = jnp.concatenate([ho_ref[0], ho_ref[1]], axis=1)
    hn = jnp.concatenate([hn_ref[0], hn_ref[1]], axis=1)
    pre = (dvo_ref[...] * (so + ho) + dvn_ref[...] * (sn + hn) + bc_ref[...])
    out_ref[...] = jnp.maximum(pre, 0.0)


def _run_comb(So, Sn, ho, hn, dvo, dvn, bc):
    half = pl.BlockSpec((2, NB, HD), lambda i: (0, i, 0))
    return pl.pallas_call(
        _comb_body,
        grid=(NBLK,),
        in_specs=[half, half, half, half,
                  pl.BlockSpec((NB, 1), lambda i: (i, 0)),
                  pl.BlockSpec((NB, 1), lambda i: (i, 0)),
                  pl.BlockSpec((1, D), lambda i: (0, 0))],
        out_specs=pl.BlockSpec((NB, D), lambda i: (i, 0)),
        out_shape=jax.ShapeDtypeStruct((N, D), _F32),
    )(So, Sn, ho, hn, dvo, dvn, bc)


def _segmax_body(v_ref, ids_ref, out_ref):
    v = v_ref[...]
    ids = ids_ref[...]
    neg = jnp.float32(-jnp.inf)
    boundary = (ids[1:] != ids[:-1]).astype(_I32)
    f = jnp.concatenate([jnp.ones((1, 1), _I32), boundary], axis=0)
    k = 1
    while k < N:
        vs = jnp.concatenate([jnp.full((k, D), neg), v[:-k]], axis=0)
        fs = jnp.concatenate([jnp.zeros((k, 1), _I32), f[:-k]], axis=0)
        v = jnp.where(f > 0, v, jnp.maximum(v, vs))
        f = jnp.bitwise_or(f, fs)
        k *= 2
    ends = jnp.concatenate([boundary, jnp.ones((1, 1), _I32)], axis=0)
    gids = lax.broadcasted_iota(_I32, (1, G), 1)
    onehotf = (ids == gids).astype(_F32)
    sel = ends.astype(_F32) * onehotf
    outv = lax.dot_general(sel, v, (((0,), (0,)), ((), ())),
                           precision=_HIGH, preferred_element_type=_F32)
    cnt = lax.dot_general(onehotf, jnp.ones((N, 1), _F32),
                          (((0,), (0,)), ((), ())),
                          precision=_HIGH, preferred_element_type=_F32)
    out_ref[...] = jnp.where(cnt == 0.0, neg, outv)


def _run_segmax(v, ids2):
    return pl.pallas_call(
        _segmax_body,
        in_specs=[pl.BlockSpec((N, D), lambda: (0, 0)),
                  pl.BlockSpec((N, 1), lambda: (0, 0))],
        out_specs=pl.BlockSpec((G, D), lambda: (0, 0)),
        out_shape=jax.ShapeDtypeStruct((G, D), _F32),
    )(v, ids2)


def kernel(x, edge_index, edge_weight, batch, edge_index_neighbor,
           edge_weight_neighbor, batch_neighbor,
           Wo0, bo0, Wn0, bn0, Wf0, bf0,
           Wo1, bo1, Wn1, bn1, Wf1, bf1):
    srco = edge_index[0]
    dsto = edge_index[1]
    ewo = edge_weight
    srcn = edge_index_neighbor[0]
    dstn = edge_index_neighbor[1]
    ewn = edge_weight_neighbor
    dstf = jnp.concatenate([dsto, dstn])
    ewf = jnp.concatenate([ewo, ewn])
    zN = jnp.zeros((N,), _F32)
    z2d = jnp.zeros((N, HD), _F32)
    ids2 = batch.reshape(N, 1)

    degp = _run_deg(dstf, ewf, zN)
    dinv = pl.pallas_call(
        _prepd_body,
        grid=(2,),
        in_specs=[pl.BlockSpec((1, NS, N // 8, 8), lambda c: (c, 0, 0, 0))],
        out_specs=pl.BlockSpec((1, N // 8, 8), lambda c: (c, 0, 0)),
        out_shape=jax.ShapeDtypeStruct((2, N // 8, 8), _F32),
    )(degp.reshape(2, NS, N // 8, 8))
    dinv = dinv.reshape(2, N)
    dvo = dinv[0].reshape(N, 1)
    dvn = dinv[1].reshape(N, 1)

    wco0, wcn0, bc0, wco1, wcn1, bc1 = pl.pallas_call(
        _prepw_body,
        out_shape=[jax.ShapeDtypeStruct((D, D), _F32),
                   jax.ShapeDtypeStruct((D, D), _F32),
                   jax.ShapeDtypeStruct((1, D), _F32)] * 2,
    )(Wo0, Wn0, Wf0, bo0.reshape(1, D), bn0.reshape(1, D), bf0.reshape(1, D),
      Wo1, Wn1, Wf1, bo1.reshape(1, D), bn1.reshape(1, D), bf1.reshape(1, D))

    embs = []
    xin = x
    for (wco, wcn, bc) in ((wco0, wcn0, bc0), (wco1, wcn1, bc1)):
        ho, hn = _run_mm(xin, wco, wcn, dvo, dvn)
        So, Sn = _run_agg(ho.reshape(2 * N, HD), hn.reshape(2 * N, HD),
                          srco, dsto, ewo, srcn, dstn, ewn, z2d)
        xin = _run_comb(So.reshape(2, N, HD), Sn.reshape(2, N, HD),
                        ho, hn, dvo, dvn, bc)
        embs.append(_run_segmax(xin, ids2))
    return tuple(embs)


# speedup vs baseline: 6.4040x; 1.3976x over previous
import functools

import jax
import jax.numpy as jnp
from jax import lax
from jax.experimental import pallas as pl
from jax.experimental.pallas import tpu as pltpu
from jax.experimental.pallas import tpu_sc as plsc

N = 10000
E = 320000
D = 128
HD = 64
G = 128
NS = 16
EPT = E // NS
CH = 2000
ROWS = E // 128
SEG = N // NS

_F32 = jnp.float32
_I32 = jnp.int32
_HIGH = lax.Precision.HIGHEST


def _dot(a, b):
    return lax.dot_general(a, b, (((1,), (0,)), ((), ())),
                           precision=_HIGH, preferred_element_type=_F32)


def _deg_body(dstf, ewf, zN, degp, deg_v, dbuf, wbuf):
    c = lax.axis_index("c")
    t = lax.axis_index("s")
    pltpu.sync_copy(zN, deg_v)
    e0 = c * E + t * EPT

    def chunk(k, _):
        base = e0 + k * CH
        pltpu.sync_copy(dstf.at[pl.ds(base, CH)], dbuf)
        pltpu.sync_copy(ewf.at[pl.ds(base, CH)], wbuf)

        def inner(i, _):
            idx = dbuf[pl.ds(i * 16, 16)]
            w = wbuf[pl.ds(i * 16, 16)]
            plsc.addupdate_scatter(deg_v, [idx], w)
            return 0

        lax.fori_loop(0, CH // 16, inner, 0)
        return 0

    lax.fori_loop(0, EPT // CH, chunk, 0)
    pltpu.sync_copy(deg_v, degp.at[pl.ds(pl.multiple_of((c * NS + t) * N, 8), N)])


PADR = 2560
RPT = PADR // NS
RC = 16
NCK = RPT // RC


def _agg_body(ho2, hn2, srco, dsto, ewo, srcn, dstn, ewn, z2d, So, Sn,
              So_sp, Sn_sp, srcb, adjb, dstb, ewb, bufA, bufB,
              gsemA, gsemB, ssemA, ssemB):
    c = lax.axis_index("c")
    t = lax.axis_index("s")
    cbase = pl.multiple_of(c * N, 8)
    bufs = (bufA, bufB)
    gsems = (gsemA, gsemB)
    ssems = (ssemA, ssemB)

    @pl.when(t == 0)
    def _():
        pltpu.sync_copy(z2d, So_sp)
        pltpu.sync_copy(z2d, Sn_sp)

    plsc.subcore_barrier()

    for (src, dst, ew, tab, S) in ((srco, dsto, ewo, ho2, So_sp),
                                   (srcn, dstn, ewn, hn2, Sn_sp)):
        def chunk_body(cb, _, src=src, dst=dst, ew=ew, tab=tab, S=S):
            r0 = pl.multiple_of(t * RPT + cb * RC, RC)
            pltpu.sync_copy(src.at[pl.ds(r0, RC)], srcb)
            pltpu.sync_copy(dst.at[pl.ds(r0, RC)], dstb)
            pltpu.sync_copy(ew.at[pl.ds(r0, RC)], ewb)
            for j in range(RC):
                for k in range(8):
                    adjb[j, pl.ds(k * 16, 16)] = srcb[j, pl.ds(k * 16, 16)] + cbase

            gd, sd = {}, {}

            def start_gather(j):
                gd[j] = pltpu.async_copy(tab.at[adjb.at[j]], bufs[j % 2],
                                         gsems[j % 2])

            def start_scatter(j):
                sd[j] = pltpu.async_copy(bufs[j % 2], S.at[dstb.at[j]],
                                         ssems[j % 2], add=True)

            start_gather(0)
            for j in range(RC):
                if j < RC - 1:
                    if j >= 1:
                        sd[j - 1].wait()
                    start_gather(j + 1)
                gd[j].wait()
                buf = bufs[j % 2]

                def scale(e, _, buf=buf, j=j):
                    er = jnp.full((16,), e, _I32)
                    bw = plsc.load_gather(ewb, [jnp.full((16,), j, _I32), er])
                    for f in range(4):
                        col = jnp.arange(16, dtype=_I32) + f * 16
                        v = plsc.load_gather(buf, [er, col])
                        plsc.store_scatter(buf, [er, col], v * bw)
                    return 0

                lax.fori_loop(0, 128, scale, 0)
                start_scatter(j)
            sd[RC - 2].wait()
            sd[RC - 1].wait()
            return 0

        lax.fori_loop(0, NCK, chunk_body, 0)

    plsc.subcore_barrier()

    @pl.when(t == 0)
    def _():
        pltpu.sync_copy(So_sp, So.at[pl.ds(cbase, N)])
        pltpu.sync_copy(Sn_sp, Sn.at[pl.ds(cbase, N)])


def _sc_mesh():
    return plsc.VectorSubcoreMesh(core_axis_name="c", subcore_axis_name="s")


def _run_deg(dstf, ewf, zN):
    return pl.kernel(
        _deg_body,
        out_type=jax.ShapeDtypeStruct((2 * NS * N,), _F32),
        mesh=_sc_mesh(),
        scratch_types=[
            pltpu.VMEM((N,), _F32),
            pltpu.VMEM((CH,), _I32),
            pltpu.VMEM((CH,), _F32),
        ],
        compiler_params=pltpu.CompilerParams(needs_layout_passes=False, use_tc_tiling_on_sc=False),
        name="gcn_deg",
    )(dstf, ewf, zN)


def _run_agg(ho2, hn2, srco, dsto, ewo, srcn, dstn, ewn, z2d):
    return pl.kernel(
        _agg_body,
        out_type=(jax.ShapeDtypeStruct((2 * N, HD), _F32),
                  jax.ShapeDtypeStruct((2 * N, HD), _F32)),
        mesh=_sc_mesh(),
        scratch_types=[
            pltpu.VMEM_SHARED((N, HD), _F32),
            pltpu.VMEM_SHARED((N, HD), _F32),
            pltpu.VMEM((RC, 128), _I32),
            pltpu.VMEM((RC, 128), _I32),
            pltpu.VMEM((RC, 128), _I32),
            pltpu.VMEM((RC, 128), _F32),
            pltpu.VMEM((128, HD), _F32),
            pltpu.VMEM((128, HD), _F32),
            pltpu.SemaphoreType.DMA,
            pltpu.SemaphoreType.DMA,
            pltpu.SemaphoreType.DMA,
            pltpu.SemaphoreType.DMA,
        ],
        compiler_params=pltpu.CompilerParams(needs_layout_passes=False, use_tc_tiling_on_sc=False),
        name="gcn_agg",
    )(ho2, hn2, srco, dsto, ewo, srcn, dstn, ewn, z2d)


def _prepw_body(wo0, wn0, wf0, bo0, bn0, bf0, wo1, wn1, wf1, bo1, bn1, bf1,
                wco0, wcn0, bc0, wco1, wcn1, bc1):
    for (wo, wn, wf, bo, bn, bf, wco, wcn, bc) in (
            (wo0, wn0, wf0, bo0, bn0, bf0, wco0, wcn0, bc0),
            (wo1, wn1, wf1, bo1, bn1, bf1, wco1, wcn1, bc1)):
        wfv = wf[...]
        wft = wfv[:D, :]
        wfb = wfv[D:, :]
        wco[...] = _dot(wo[...], wft)
        wcn[...] = _dot(wn[...], wfb)
        bc[...] = _dot(bo[...], wft) + _dot(bn[...], wfb) + bf[...]


def _prepd_body(degp_ref, dinv_ref):
    s = jnp.sum(degp_ref[...], axis=(0, 1))
    dinv_ref[...] = lax.rsqrt(1.0 + s)[None]


NB = 400
NBLK = N // NB


def _mm_body(x_ref, wo_ref, wn_ref, dvo_ref, dvn_ref, ho_ref, hn_ref):
    xb = x_ref[...]
    ho = _dot(xb, wo_ref[...]) * dvo_ref[...]
    hn = _dot(xb, wn_ref[...]) * dvn_ref[...]
    ho_ref[...] = jnp.stack([ho[:, :HD], ho[:, HD:]], axis=0)
    hn_ref[...] = jnp.stack([hn[:, :HD], hn[:, HD:]], axis=0)


def _run_mm(xin, wco, wcn, dvo, dvn):
    return pl.pallas_call(
        _mm_body,
        grid=(NBLK,),
        in_specs=[
            pl.BlockSpec((NB, D), lambda i: (i, 0)),
            pl.BlockSpec((D, D), lambda i: (0, 0)),
            pl.BlockSpec((D, D), lambda i: (0, 0)),
            pl.BlockSpec((NB, 1), lambda i: (i, 0)),
            pl.BlockSpec((NB, 1), lambda i: (i, 0)),
        ],
        out_specs=[
            pl.BlockSpec((2, NB, HD), lambda i: (0, i, 0)),
            pl.BlockSpec((2, NB, HD), lambda i: (0, i, 0)),
        ],
        out_shape=[jax.ShapeDtypeStruct((2, N, HD), _F32)] * 2,
    )(xin, wco, wcn, dvo, dvn)


def _comb_body(so_ref, sn_ref, ho_ref, hn_ref, dvo_ref, dvn_ref, bc_ref,
               out_ref):
    so = jnp.concatenate([so_ref[0], so_ref[1]], axis=1)
    sn = jnp.concatenate([sn_ref[0], sn_ref[1]], axis=1)
    ho = jnp.concatenate([ho_ref[0], ho_ref[1]], axis=1)
    hn = jnp.concatenate([hn_ref[0], hn_ref[1]], axis=1)
    pre = (dvo_ref[...] * (so + ho) + dvn_ref[...] * (sn + hn) + bc_ref[...])
    out_ref[...] = jnp.maximum(pre, 0.0)


def _run_comb(So, Sn, ho, hn, dvo, dvn, bc):
    half = pl.BlockSpec((2, NB, HD), lambda i: (0, i, 0))
    return pl.pallas_call(
        _comb_body,
        grid=(NBLK,),
        in_specs=[half, half, half, half,
                  pl.BlockSpec((NB, 1), lambda i: (i, 0)),
                  pl.BlockSpec((NB, 1), lambda i: (i, 0)),
                  pl.BlockSpec((1, D), lambda i: (0, 0))],
        out_specs=pl.BlockSpec((NB, D), lambda i: (i, 0)),
        out_shape=jax.ShapeDtypeStruct((N, D), _F32),
    )(So, Sn, ho, hn, dvo, dvn, bc)


def _segmax_body(v_ref, ids_ref, out_ref):
    v = v_ref[...]
    ids = ids_ref[...]
    neg = jnp.float32(-jnp.inf)
    boundary = (ids[1:] != ids[:-1]).astype(_I32)
    f = jnp.concatenate([jnp.ones((1, 1), _I32), boundary], axis=0)
    k = 1
    while k < N:
        vs = jnp.concatenate([jnp.full((k, D), neg), v[:-k]], axis=0)
        fs = jnp.concatenate([jnp.zeros((k, 1), _I32), f[:-k]], axis=0)
        v = jnp.where(f > 0, v, jnp.maximum(v, vs))
        f = jnp.bitwise_or(f, fs)
        k *= 2
    ends = jnp.concatenate([boundary, jnp.ones((1, 1), _I32)], axis=0)
    gids = lax.broadcasted_iota(_I32, (1, G), 1)
    onehotf = (ids == gids).astype(_F32)
    sel = ends.astype(_F32) * onehotf
    outv = lax.dot_general(sel, v, (((0,), (0,)), ((), ())),
                           precision=_HIGH, preferred_element_type=_F32)
    cnt = lax.dot_general(onehotf, jnp.ones((N, 1), _F32),
                          (((0,), (0,)), ((), ())),
                          precision=_HIGH, preferred_element_type=_F32)
    out_ref[...] = jnp.where(cnt == 0.0, neg, outv)


def _run_segmax(v, ids2):
    return pl.pallas_call(
        _segmax_body,
        in_specs=[pl.BlockSpec((N, D), lambda: (0, 0)),
                  pl.BlockSpec((N, 1), lambda: (0, 0))],
        out_specs=pl.BlockSpec((G, D), lambda: (0, 0)),
        out_shape=jax.ShapeDtypeStruct((G, D), _F32),
    )(v, ids2)


def kernel(x, edge_index, edge_weight, batch, edge_index_neighbor,
           edge_weight_neighbor, batch_neighbor,
           Wo0, bo0, Wn0, bn0, Wf0, bf0,
           Wo1, bo1, Wn1, bn1, Wf1, bf1):
    dstf = jnp.concatenate([edge_index[1], edge_index_neighbor[1]])
    ewf = jnp.concatenate([edge_weight, edge_weight_neighbor])

    pe = PADR * 128 - E
    zi = jnp.zeros((pe,), _I32)
    zf = jnp.zeros((pe,), _F32)
    srco = jnp.concatenate([edge_index[0], zi]).reshape(PADR, 128)
    dsto = jnp.concatenate([edge_index[1], zi]).reshape(PADR, 128)
    ewo = jnp.concatenate([edge_weight, zf]).reshape(PADR, 128)
    srcn = jnp.concatenate([edge_index_neighbor[0], zi]).reshape(PADR, 128)
    dstn = jnp.concatenate([edge_index_neighbor[1], zi]).reshape(PADR, 128)
    ewn = jnp.concatenate([edge_weight_neighbor, zf]).reshape(PADR, 128)
    zN = jnp.zeros((N,), _F32)
    z2d = jnp.zeros((N, HD), _F32)
    ids2 = batch.reshape(N, 1)

    degp = _run_deg(dstf, ewf, zN)
    dinv = pl.pallas_call(
        _prepd_body,
        grid=(2,),
        in_specs=[pl.BlockSpec((1, NS, N // 8, 8), lambda c: (c, 0, 0, 0))],
        out_specs=pl.BlockSpec((1, N // 8, 8), lambda c: (c, 0, 0)),
        out_shape=jax.ShapeDtypeStruct((2, N // 8, 8), _F32),
    )(degp.reshape(2, NS, N // 8, 8))
    dinv = dinv.reshape(2, N)
    dvo = dinv[0].reshape(N, 1)
    dvn = dinv[1].reshape(N, 1)

    wco0, wcn0, bc0, wco1, wcn1, bc1 = pl.pallas_call(
        _prepw_body,
        out_shape=[jax.ShapeDtypeStruct((D, D), _F32),
                   jax.ShapeDtypeStruct((D, D), _F32),
                   jax.ShapeDtypeStruct((1, D), _F32)] * 2,
    )(Wo0, Wn0, Wf0, bo0.reshape(1, D), bn0.reshape(1, D), bf0.reshape(1, D),
      Wo1, Wn1, Wf1, bo1.reshape(1, D), bn1.reshape(1, D), bf1.reshape(1, D))

    embs = []
    xin = x
    for (wco, wcn, bc) in ((wco0, wcn0, bc0), (wco1, wcn1, bc1)):
        ho, hn = _run_mm(xin, wco, wcn, dvo, dvn)
        So, Sn = _run_agg(ho.reshape(2 * N, HD), hn.reshape(2 * N, HD),
                          srco, dsto, ewo, srcn, dstn, ewn, z2d)
        xin = _run_comb(So.reshape(2, N, HD), Sn.reshape(2, N, HD),
                        ho, hn, dvo, dvn, bc)
        embs.append(_run_segmax(xin, ids2))
    return tuple(embs)


# speedup vs baseline: 9.7938x; 1.5293x over previous
import functools

import jax
import jax.numpy as jnp
from jax import lax
from jax.experimental import pallas as pl
from jax.experimental.pallas import tpu as pltpu
from jax.experimental.pallas import tpu_sc as plsc

N = 10000
E = 320000
D = 128
HD = 64
G = 128
NS = 16
EPT = E // NS
CH = 2000
ROWS = E // 128
SEG = N // NS

_F32 = jnp.float32
_I32 = jnp.int32
_HIGH = lax.Precision.HIGHEST


def _dot(a, b):
    return lax.dot_general(a, b, (((1,), (0,)), ((), ())),
                           precision=_HIGH, preferred_element_type=_F32)


def _deg_body(dstf, ewf, zN, degp, deg_v, dbuf, wbuf):
    c = lax.axis_index("c")
    t = lax.axis_index("s")
    pltpu.sync_copy(zN, deg_v)
    e0 = c * E + t * EPT

    def chunk(k, _):
        base = e0 + k * CH
        pltpu.sync_copy(dstf.at[pl.ds(base, CH)], dbuf)
        pltpu.sync_copy(ewf.at[pl.ds(base, CH)], wbuf)

        def inner(i, _):
            idx = dbuf[pl.ds(i * 16, 16)]
            w = wbuf[pl.ds(i * 16, 16)]
            plsc.addupdate_scatter(deg_v, [idx], w)
            return 0

        lax.fori_loop(0, CH // 16, inner, 0)
        return 0

    lax.fori_loop(0, EPT // CH, chunk, 0)
    pltpu.sync_copy(deg_v, degp.at[pl.ds(pl.multiple_of((c * NS + t) * N, 8), N)])


PADR = 2560
RPT = PADR // NS
RC = 16
NCK = RPT // RC


def _agg_body(ho2, hn2, srco, dsto, ewo, srcn, dstn, ewn, z2d, So, Sn,
              So_sp, Sn_sp, srcb, adjb, dstb, ewb, bufA, bufB,
              gsemA, gsemB, ssemA, ssemB):
    c = lax.axis_index("c")
    t = lax.axis_index("s")
    cbase = pl.multiple_of(c * N, 8)
    bufs = (bufA, bufB)
    gsems = (gsemA, gsemB)
    ssems = (ssemA, ssemB)

    @pl.when(t == 0)
    def _():
        pltpu.sync_copy(z2d, So_sp)
        pltpu.sync_copy(z2d, Sn_sp)

    plsc.subcore_barrier()

    for (src, dst, ew, tab, S) in ((srco, dsto, ewo, ho2, So_sp),
                                   (srcn, dstn, ewn, hn2, Sn_sp)):
        def chunk_body(cb, _, src=src, dst=dst, ew=ew, tab=tab, S=S):
            r0 = pl.multiple_of(t * RPT + cb * RC, RC)
            pltpu.sync_copy(src.at[pl.ds(r0, RC)], srcb)
            pltpu.sync_copy(dst.at[pl.ds(r0, RC)], dstb)
            pltpu.sync_copy(ew.at[pl.ds(r0, RC)], ewb)
            for j in range(RC):
                for k in range(8):
                    adjb[j, pl.ds(k * 16, 16)] = srcb[j, pl.ds(k * 16, 16)] + cbase

            gd, sd = {}, {}

            def start_gather(j):
                gd[j] = pltpu.async_copy(tab.at[adjb.at[j]], bufs[j % 2],
                                         gsems[j % 2])

            def start_scatter(j):
                sd[j] = pltpu.async_copy(bufs[j % 2], S.at[dstb.at[j]],
                                         ssems[j % 2], add=True)

            start_gather(0)
            for j in range(RC):
                if j < RC - 1:
                    if j >= 1:
                        sd[j - 1].wait()
                    start_gather(j + 1)
                gd[j].wait()
                buf = bufs[j % 2]

                @plsc.parallel_loop(0, 128, unroll=8)
                def scale(e, buf=buf, j=j):
                    er = jnp.full((16,), e, _I32)
                    bw = plsc.load_gather(ewb, [jnp.full((16,), j, _I32), er])
                    for f in range(4):
                        sl = pl.ds(f * 16, 16)
                        buf[e, sl] = buf[e, sl] * bw

                start_scatter(j)
            sd[RC - 2].wait()
            sd[RC - 1].wait()
            return 0

        lax.fori_loop(0, NCK, chunk_body, 0)

    plsc.subcore_barrier()

    @pl.when(t == 0)
    def _():
        pltpu.sync_copy(So_sp, So.at[pl.ds(cbase, N)])
        pltpu.sync_copy(Sn_sp, Sn.at[pl.ds(cbase, N)])


def _sc_mesh():
    return plsc.VectorSubcoreMesh(core_axis_name="c", subcore_axis_name="s")


def _run_deg(dstf, ewf, zN):
    return pl.kernel(
        _deg_body,
        out_type=jax.ShapeDtypeStruct((2 * NS * N,), _F32),
        mesh=_sc_mesh(),
        scratch_types=[
            pltpu.VMEM((N,), _F32),
            pltpu.VMEM((CH,), _I32),
            pltpu.VMEM((CH,), _F32),
        ],
        compiler_params=pltpu.CompilerParams(needs_layout_passes=False, use_tc_tiling_on_sc=False),
        name="gcn_deg",
    )(dstf, ewf, zN)


def _run_agg(ho2, hn2, srco, dsto, ewo, srcn, dstn, ewn, z2d):
    return pl.kernel(
        _agg_body,
        out_type=(jax.ShapeDtypeStruct((2 * N, HD), _F32),
                  jax.ShapeDtypeStruct((2 * N, HD), _F32)),
        mesh=_sc_mesh(),
        scratch_types=[
            pltpu.VMEM_SHARED((N, HD), _F32),
            pltpu.VMEM_SHARED((N, HD), _F32),
            pltpu.VMEM((RC, 128), _I32),
            pltpu.VMEM((RC, 128), _I32),
            pltpu.VMEM((RC, 128), _I32),
            pltpu.VMEM((RC, 128), _F32),
            pltpu.VMEM((128, HD), _F32),
            pltpu.VMEM((128, HD), _F32),
            pltpu.SemaphoreType.DMA,
            pltpu.SemaphoreType.DMA,
            pltpu.SemaphoreType.DMA,
            pltpu.SemaphoreType.DMA,
        ],
        compiler_params=pltpu.CompilerParams(needs_layout_passes=False, use_tc_tiling_on_sc=False),
        name="gcn_agg",
    )(ho2, hn2, srco, dsto, ewo, srcn, dstn, ewn, z2d)


def _prepw_body(wo0, wn0, wf0, bo0, bn0, bf0, wo1, wn1, wf1, bo1, bn1, bf1,
                wco0, wcn0, bc0, wco1, wcn1, bc1):
    for (wo, wn, wf, bo, bn, bf, wco, wcn, bc) in (
            (wo0, wn0, wf0, bo0, bn0, bf0, wco0, wcn0, bc0),
            (wo1, wn1, wf1, bo1, bn1, bf1, wco1, wcn1, bc1)):
        wfv = wf[...]
        wft = wfv[:D, :]
        wfb = wfv[D:, :]
        wco[...] = _dot(wo[...], wft)
        wcn[...] = _dot(wn[...], wfb)
        bc[...] = _dot(bo[...], wft) + _dot(bn[...], wfb) + bf[...]


def _prepd_body(degp_ref, dinv_ref):
    s = jnp.sum(degp_ref[...], axis=(0, 1))
    dinv_ref[...] = lax.rsqrt(1.0 + s)[None]


NB = 400
NBLK = N // NB


def _mm_body(x_ref, wo_ref, wn_ref, dvo_ref, dvn_ref, ho_ref, hn_ref):
    xb = x_ref[...]
    ho = _dot(xb, wo_ref[...]) * dvo_ref[...]
    hn = _dot(xb, wn_ref[...]) * dvn_ref[...]
    ho_ref[...] = jnp.stack([ho[:, :HD], ho[:, HD:]], axis=0)
    hn_ref[...] = jnp.stack([hn[:, :HD], hn[:, HD:]], axis=0)


def _run_mm(xin, wco, wcn, dvo, dvn):
    return pl.pallas_call(
        _mm_body,
        grid=(NBLK,),
        in_specs=[
            pl.BlockSpec((NB, D), lambda i: (i, 0)),
            pl.BlockSpec((D, D), lambda i: (0, 0)),
            pl.BlockSpec((D, D), lambda i: (0, 0)),
            pl.BlockSpec((NB, 1), lambda i: (i, 0)),
            pl.BlockSpec((NB, 1), lambda i: (i, 0)),
        ],
        out_specs=[
            pl.BlockSpec((2, NB, HD), lambda i: (0, i, 0)),
            pl.BlockSpec((2, NB, HD), lambda i: (0, i, 0)),
        ],
        out_shape=[jax.ShapeDtypeStruct((2, N, HD), _F32)] * 2,
    )(xin, wco, wcn, dvo, dvn)


def _comb_body(so_ref, sn_ref, ho_ref, hn_ref, dvo_ref, dvn_ref, bc_ref,
               out_ref):
    so = jnp.concatenate([so_ref[0], so_ref[1]], axis=1)
    sn = jnp.concatenate([sn_ref[0], sn_ref[1]], axis=1)
    ho = jnp.concatenate([ho_ref[0], ho_ref[1]], axis=1)
    hn = jnp.concatenate([hn_ref[0], hn_ref[1]], axis=1)
    pre = (dvo_ref[...] * (so + ho) + dvn_ref[...] * (sn + hn) + bc_ref[...])
    out_ref[...] = jnp.maximum(pre, 0.0)


def _run_comb(So, Sn, ho, hn, dvo, dvn, bc):
    half = pl.BlockSpec((2, NB, HD), lambda i: (0, i, 0))
    return pl.pallas_call(
        _comb_body,
        grid=(NBLK,),
        in_specs=[half, half, half, half,
                  pl.BlockSpec((NB, 1), lambda i: (i, 0)),
                  pl.BlockSpec((NB, 1), lambda i: (i, 0)),
                  pl.BlockSpec((1, D), lambda i: (0, 0))],
        out_specs=pl.BlockSpec((NB, D), lambda i: (i, 0)),
        out_shape=jax.ShapeDtypeStruct((N, D), _F32),
    )(So, Sn, ho, hn, dvo, dvn, bc)


def _segmax_body(v_ref, ids_ref, out_ref):
    v = v_ref[...]
    ids = ids_ref[...]
    neg = jnp.float32(-jnp.inf)
    boundary = (ids[1:] != ids[:-1]).astype(_I32)
    f = jnp.concatenate([jnp.ones((1, 1), _I32), boundary], axis=0)
    k = 1
    while k < N:
        vs = jnp.concatenate([jnp.full((k, D), neg), v[:-k]], axis=0)
        fs = jnp.concatenate([jnp.zeros((k, 1), _I32), f[:-k]], axis=0)
        v = jnp.where(f > 0, v, jnp.maximum(v, vs))
        f = jnp.bitwise_or(f, fs)
        k *= 2
    ends = jnp.concatenate([boundary, jnp.ones((1, 1), _I32)], axis=0)
    gids = lax.broadcasted_iota(_I32, (1, G), 1)
    onehotf = (ids == gids).astype(_F32)
    sel = ends.astype(_F32) * onehotf
    outv = lax.dot_general(sel, v, (((0,), (0,)), ((), ())),
                           precision=_HIGH, preferred_element_type=_F32)
    cnt = lax.dot_general(onehotf, jnp.ones((N, 1), _F32),
                          (((0,), (0,)), ((), ())),
                          precision=_HIGH, preferred_element_type=_F32)
    out_ref[...] = jnp.where(cnt == 0.0, neg, outv)


def _run_segmax(v, ids2):
    return pl.pallas_call(
        _segmax_body,
        in_specs=[pl.BlockSpec((N, D), lambda: (0, 0)),
                  pl.BlockSpec((N, 1), lambda: (0, 0))],
        out_specs=pl.BlockSpec((G, D), lambda: (0, 0)),
        out_shape=jax.ShapeDtypeStruct((G, D), _F32),
    )(v, ids2)


def kernel(x, edge_index, edge_weight, batch, edge_index_neighbor,
           edge_weight_neighbor, batch_neighbor,
           Wo0, bo0, Wn0, bn0, Wf0, bf0,
           Wo1, bo1, Wn1, bn1, Wf1, bf1):
    dstf = jnp.concatenate([edge_index[1], edge_index_neighbor[1]])
    ewf = jnp.concatenate([edge_weight, edge_weight_neighbor])

    pe = PADR * 128 - E
    zi = jnp.zeros((pe,), _I32)
    zf = jnp.zeros((pe,), _F32)
    srco = jnp.concatenate([edge_index[0], zi]).reshape(PADR, 128)
    dsto = jnp.concatenate([edge_index[1], zi]).reshape(PADR, 128)
    ewo = jnp.concatenate([edge_weight, zf]).reshape(PADR, 128)
    srcn = jnp.concatenate([edge_index_neighbor[0], zi]).reshape(PADR, 128)
    dstn = jnp.concatenate([edge_index_neighbor[1], zi]).reshape(PADR, 128)
    ewn = jnp.concatenate([edge_weight_neighbor, zf]).reshape(PADR, 128)
    zN = jnp.zeros((N,), _F32)
    z2d = jnp.zeros((N, HD), _F32)
    ids2 = batch.reshape(N, 1)

    degp = _run_deg(dstf, ewf, zN)
    dinv = pl.pallas_call(
        _prepd_body,
        grid=(2,),
        in_specs=[pl.BlockSpec((1, NS, N // 8, 8), lambda c: (c, 0, 0, 0))],
        out_specs=pl.BlockSpec((1, N // 8, 8), lambda c: (c, 0, 0)),
        out_shape=jax.ShapeDtypeStruct((2, N // 8, 8), _F32),
    )(degp.reshape(2, NS, N // 8, 8))
    dinv = dinv.reshape(2, N)
    dvo = dinv[0].reshape(N, 1)
    dvn = dinv[1].reshape(N, 1)

    wco0, wcn0, bc0, wco1, wcn1, bc1 = pl.pallas_call(
        _prepw_body,
        out_shape=[jax.ShapeDtypeStruct((D, D), _F32),
                   jax.ShapeDtypeStruct((D, D), _F32),
                   jax.ShapeDtypeStruct((1, D), _F32)] * 2,
    )(Wo0, Wn0, Wf0, bo0.reshape(1, D), bn0.reshape(1, D), bf0.reshape(1, D),
      Wo1, Wn1, Wf1, bo1.reshape(1, D), bn1.reshape(1, D), bf1.reshape(1, D))

    embs = []
    xin = x
    for (wco, wcn, bc) in ((wco0, wcn0, bc0), (wco1, wcn1, bc1)):
        ho, hn = _run_mm(xin, wco, wcn, dvo, dvn)
        So, Sn = _run_agg(ho.reshape(2 * N, HD), hn.reshape(2 * N, HD),
                          srco, dsto, ewo, srcn, dstn, ewn, z2d)
        xin = _run_comb(So.reshape(2, N, HD), Sn.reshape(2, N, HD),
                        ho, hn, dvo, dvn, bc)
        embs.append(_run_segmax(xin, ids2))
    return tuple(embs)


# speedup vs baseline: 10.3217x; 1.0539x over previous
import functools

import jax
import jax.numpy as jnp
from jax import lax
from jax.experimental import pallas as pl
from jax.experimental.pallas import tpu as pltpu
from jax.experimental.pallas import tpu_sc as plsc

N = 10000
E = 320000
D = 128
HD = 64
G = 128
NS = 16
EPT = E // NS
CH = 2000
ROWS = E // 128
SEG = N // NS

_F32 = jnp.float32
_I32 = jnp.int32
_HIGH = lax.Precision.HIGHEST


def _dot(a, b):
    return lax.dot_general(a, b, (((1,), (0,)), ((), ())),
                           precision=_HIGH, preferred_element_type=_F32)


def _deg_body(dstf, ewf, zN, degp, deg_v, dbuf, wbuf):
    c = lax.axis_index("c")
    t = lax.axis_index("s")
    pltpu.sync_copy(zN, deg_v)
    e0 = c * E + t * EPT

    def chunk(k, _):
        base = e0 + k * CH
        pltpu.sync_copy(dstf.at[pl.ds(base, CH)], dbuf)
        pltpu.sync_copy(ewf.at[pl.ds(base, CH)], wbuf)

        def inner(i, _):
            idx = dbuf[pl.ds(i * 16, 16)]
            w = wbuf[pl.ds(i * 16, 16)]
            plsc.addupdate_scatter(deg_v, [idx], w)
            return 0

        lax.fori_loop(0, CH // 16, inner, 0)
        return 0

    lax.fori_loop(0, EPT // CH, chunk, 0)
    pltpu.sync_copy(deg_v, degp.at[pl.ds(pl.multiple_of((c * NS + t) * N, 8), N)])


PADR = 2560
RPT = PADR // NS
NBUF = 4
CHR = 40
NCH = RPT // CHR


def _agg_body(ho2, hn2, srco, dsto, ewo, srcn, dstn, ewn, z2d, So, Sn,
              So_sp, Sn_sp, srcb, dstb, ewb, buf0, buf1, buf2, buf3,
              gsem0, gsem1, gsem2, gsem3, ssem0, ssem1, ssem2, ssem3):
    c = lax.axis_index("c")
    t = lax.axis_index("s")
    cbase = pl.multiple_of(c * N, 8)
    bufs = (buf0, buf1, buf2, buf3)
    gsems = (gsem0, gsem1, gsem2, gsem3)
    ssems = (ssem0, ssem1, ssem2, ssem3)

    @pl.when(t == 0)
    def _():
        pltpu.sync_copy(z2d, So_sp)
        pltpu.sync_copy(z2d, Sn_sp)

    plsc.subcore_barrier()
    r0 = pl.multiple_of(t * RPT, RPT)

    for (src, dst, ew, tab, S) in ((srco, dsto, ewo, ho2, So_sp),
                                   (srcn, dstn, ewn, hn2, Sn_sp)):
        def sg_start(r, k, tab=tab):
            pltpu.async_copy(tab.at[srcb.at[r]], bufs[k], gsems[k])

        def sg_wait(k, tab=tab):
            pltpu.make_async_copy(tab.at[srcb.at[0]], bufs[k], gsems[k]).wait()

        def ss_start(r, k, S=S):
            pltpu.async_copy(bufs[k], S.at[dstb.at[r]], ssems[k], add=True)

        def ss_wait(k, S=S):
            pltpu.make_async_copy(bufs[k], S.at[dstb.at[0]], ssems[k]).wait()

        def chunk_body(ch, _, src=src, dst=dst, ew=ew, tab=tab, S=S):
            rb = pl.multiple_of(r0 + ch * CHR, 8)
            pltpu.sync_copy(src.at[pl.ds(rb, CHR)], srcb)
            pltpu.sync_copy(dst.at[pl.ds(rb, CHR)], dstb)
            pltpu.sync_copy(ew.at[pl.ds(rb, CHR)], ewb)

            @plsc.parallel_loop(0, CHR, unroll=2)
            def adjust(r):
                for k in range(8):
                    sl = pl.ds(k * 16, 16)
                    srcb[r, sl] = srcb[r, sl] + cbase

            for k in range(NBUF - 1):
                sg_start(k, k)

            def q_body(q, _):
                for k in range(NBUF):
                    r = q * NBUF + k
                    kn = (k + NBUF - 1) % NBUF

                    @pl.when(jnp.logical_and(r + NBUF - 1 < CHR, r >= 1))
                    def _(kn=kn):
                        ss_wait(kn)

                    @pl.when(r + NBUF - 1 < CHR)
                    def _(r=r, kn=kn):
                        sg_start(r + NBUF - 1, kn)

                    sg_wait(k)
                    buf = bufs[k]

                    @plsc.parallel_loop(0, 128, unroll=8)
                    def scale(e, buf=buf, r=r):
                        er = jnp.full((16,), e, _I32)
                        bw = plsc.load_gather(ewb,
                                              [jnp.full((16,), r, _I32), er])
                        for f in range(4):
                            sl = pl.ds(f * 16, 16)
                            buf[e, sl] = buf[e, sl] * bw

                    ss_start(r, k)
                return 0

            lax.fori_loop(0, CHR // NBUF, q_body, 0)
            for k in range(NBUF):
                ss_wait(k)
            return 0

        lax.fori_loop(0, NCH, chunk_body, 0)

    plsc.subcore_barrier()

    @pl.when(t == 0)
    def _():
        pltpu.sync_copy(So_sp, So.at[pl.ds(cbase, N)])
        pltpu.sync_copy(Sn_sp, Sn.at[pl.ds(cbase, N)])


def _sc_mesh():
    return plsc.VectorSubcoreMesh(core_axis_name="c", subcore_axis_name="s")


def _run_deg(dstf, ewf, zN):
    return pl.kernel(
        _deg_body,
        out_type=jax.ShapeDtypeStruct((2 * NS * N,), _F32),
        mesh=_sc_mesh(),
        scratch_types=[
            pltpu.VMEM((N,), _F32),
            pltpu.VMEM((CH,), _I32),
            pltpu.VMEM((CH,), _F32),
        ],
        compiler_params=pltpu.CompilerParams(needs_layout_passes=False, use_tc_tiling_on_sc=False),
        name="gcn_deg",
    )(dstf, ewf, zN)


def _run_agg(ho2, hn2, srco, dsto, ewo, srcn, dstn, ewn, z2d):
    return pl.kernel(
        _agg_body,
        out_type=(jax.ShapeDtypeStruct((2 * N, HD), _F32),
                  jax.ShapeDtypeStruct((2 * N, HD), _F32)),
        mesh=_sc_mesh(),
        scratch_types=[
            pltpu.VMEM_SHARED((N, HD), _F32),
            pltpu.VMEM_SHARED((N, HD), _F32),
            pltpu.VMEM((CHR, 128), _I32),
            pltpu.VMEM((CHR, 128), _I32),
            pltpu.VMEM((CHR, 128), _F32),
            pltpu.VMEM((128, HD), _F32),
            pltpu.VMEM((128, HD), _F32),
            pltpu.VMEM((128, HD), _F32),
            pltpu.VMEM((128, HD), _F32),
        ] + [pltpu.SemaphoreType.DMA] * 8,
        compiler_params=pltpu.CompilerParams(needs_layout_passes=False, use_tc_tiling_on_sc=False),
        name="gcn_agg",
    )(ho2, hn2, srco, dsto, ewo, srcn, dstn, ewn, z2d)


def _prepw_body(wo0, wn0, wf0, bo0, bn0, bf0, wo1, wn1, wf1, bo1, bn1, bf1,
                wco0, wcn0, bc0, wco1, wcn1, bc1):
    for (wo, wn, wf, bo, bn, bf, wco, wcn, bc) in (
            (wo0, wn0, wf0, bo0, bn0, bf0, wco0, wcn0, bc0),
            (wo1, wn1, wf1, bo1, bn1, bf1, wco1, wcn1, bc1)):
        wfv = wf[...]
        wft = wfv[:D, :]
        wfb = wfv[D:, :]
        wco[...] = _dot(wo[...], wft)
        wcn[...] = _dot(wn[...], wfb)
        bc[...] = _dot(bo[...], wft) + _dot(bn[...], wfb) + bf[...]


def _prepd_body(degp_ref, dinv_ref):
    s = jnp.sum(degp_ref[...], axis=(0, 1))
    dinv_ref[...] = lax.rsqrt(1.0 + s)[None]


NB = 400
NBLK = N // NB


def _mm_body(x_ref, wo_ref, wn_ref, dvo_ref, dvn_ref, ho_ref, hn_ref):
    xb = x_ref[...]
    ho = _dot(xb, wo_ref[...]) * dvo_ref[...]
    hn = _dot(xb, wn_ref[...]) * dvn_ref[...]
    ho_ref[...] = jnp.stack([ho[:, :HD], ho[:, HD:]], axis=0)
    hn_ref[...] = jnp.stack([hn[:, :HD], hn[:, HD:]], axis=0)


def _run_mm(xin, wco, wcn, dvo, dvn):
    return pl.pallas_call(
        _mm_body,
        grid=(NBLK,),
        in_specs=[
            pl.BlockSpec((NB, D), lambda i: (i, 0)),
            pl.BlockSpec((D, D), lambda i: (0, 0)),
            pl.BlockSpec((D, D), lambda i: (0, 0)),
            pl.BlockSpec((NB, 1), lambda i: (i, 0)),
            pl.BlockSpec((NB, 1), lambda i: (i, 0)),
        ],
        out_specs=[
            pl.BlockSpec((2, NB, HD), lambda i: (0, i, 0)),
            pl.BlockSpec((2, NB, HD), lambda i: (0, i, 0)),
        ],
        out_shape=[jax.ShapeDtypeStruct((2, N, HD), _F32)] * 2,
    )(xin, wco, wcn, dvo, dvn)


def _comb_body(so_ref, sn_ref, ho_ref, hn_ref, dvo_ref, dvn_ref, bc_ref,
               out_ref):
    so = jnp.concatenate([so_ref[0], so_ref[1]], axis=1)
    sn = jnp.concatenate([sn_ref[0], sn_ref[1]], axis=1)
    ho = jnp.concatenate([ho_ref[0], ho_ref[1]], axis=1)
    hn = jnp.concatenate([hn_ref[0], hn_ref[1]], axis=1)
    pre = (dvo_ref[...] * (so + ho) + dvn_ref[...] * (sn + hn) + bc_ref[...])
    out_ref[...] = jnp.maximum(pre, 0.0)


def _run_comb(So, Sn, ho, hn, dvo, dvn, bc):
    half = pl.BlockSpec((2, NB, HD), lambda i: (0, i, 0))
    return pl.pallas_call(
        _comb_body,
        grid=(NBLK,),
        in_specs=[half, half, half, half,
                  pl.BlockSpec((NB, 1), lambda i: (i, 0)),
                  pl.BlockSpec((NB, 1), lambda i: (i, 0)),
                  pl.BlockSpec((1, D), lambda i: (0, 0))],
        out_specs=pl.BlockSpec((NB, D), lambda i: (i, 0)),
        out_shape=jax.ShapeDtypeStruct((N, D), _F32),
    )(So, Sn, ho, hn, dvo, dvn, bc)


def _segmax_body(v_ref, ids_ref, out_ref):
    v = v_ref[...]
    ids = ids_ref[...]
    neg = jnp.float32(-jnp.inf)
    boundary = (ids[1:] != ids[:-1]).astype(_I32)
    f = jnp.concatenate([jnp.ones((1, 1), _I32), boundary], axis=0)
    k = 1
    while k < N:
        vs = jnp.concatenate([jnp.full((k, D), neg), v[:-k]], axis=0)
        fs = jnp.concatenate([jnp.zeros((k, 1), _I32), f[:-k]], axis=0)
        v = jnp.where(f > 0, v, jnp.maximum(v, vs))
        f = jnp.bitwise_or(f, fs)
        k *= 2
    ends = jnp.concatenate([boundary, jnp.ones((1, 1), _I32)], axis=0)
    gids = lax.broadcasted_iota(_I32, (1, G), 1)
    onehotf = (ids == gids).astype(_F32)
    sel = ends.astype(_F32) * onehotf
    outv = lax.dot_general(sel, v, (((0,), (0,)), ((), ())),
                           precision=_HIGH, preferred_element_type=_F32)
    cnt = lax.dot_general(onehotf, jnp.ones((N, 1), _F32),
                          (((0,), (0,)), ((), ())),
                          precision=_HIGH, preferred_element_type=_F32)
    out_ref[...] = jnp.where(cnt == 0.0, neg, outv)


def _run_segmax(v, ids2):
    return pl.pallas_call(
        _segmax_body,
        in_specs=[pl.BlockSpec((N, D), lambda: (0, 0)),
                  pl.BlockSpec((N, 1), lambda: (0, 0))],
        out_specs=pl.BlockSpec((G, D), lambda: (0, 0)),
        out_shape=jax.ShapeDtypeStruct((G, D), _F32),
    )(v, ids2)


def kernel(x, edge_index, edge_weight, batch, edge_index_neighbor,
           edge_weight_neighbor, batch_neighbor,
           Wo0, bo0, Wn0, bn0, Wf0, bf0,
           Wo1, bo1, Wn1, bn1, Wf1, bf1):
    dstf = jnp.concatenate([edge_index[1], edge_index_neighbor[1]])
    ewf = jnp.concatenate([edge_weight, edge_weight_neighbor])

    pe = PADR * 128 - E
    zi = jnp.zeros((pe,), _I32)
    zf = jnp.zeros((pe,), _F32)
    srco = jnp.concatenate([edge_index[0], zi]).reshape(PADR, 128)
    dsto = jnp.concatenate([edge_index[1], zi]).reshape(PADR, 128)
    ewo = jnp.concatenate([edge_weight, zf]).reshape(PADR, 128)
    srcn = jnp.concatenate([edge_index_neighbor[0], zi]).reshape(PADR, 128)
    dstn = jnp.concatenate([edge_index_neighbor[1], zi]).reshape(PADR, 128)
    ewn = jnp.concatenate([edge_weight_neighbor, zf]).reshape(PADR, 128)
    zN = jnp.zeros((N,), _F32)
    z2d = jnp.zeros((N, HD), _F32)
    ids2 = batch.reshape(N, 1)

    degp = _run_deg(dstf, ewf, zN)
    dinv = pl.pallas_call(
        _prepd_body,
        grid=(2,),
        in_specs=[pl.BlockSpec((1, NS, N // 8, 8), lambda c: (c, 0, 0, 0))],
        out_specs=pl.BlockSpec((1, N // 8, 8), lambda c: (c, 0, 0)),
        out_shape=jax.ShapeDtypeStruct((2, N // 8, 8), _F32),
    )(degp.reshape(2, NS, N // 8, 8))
    dinv = dinv.reshape(2, N)
    dvo = dinv[0].reshape(N, 1)
    dvn = dinv[1].reshape(N, 1)

    wco0, wcn0, bc0, wco1, wcn1, bc1 = pl.pallas_call(
        _prepw_body,
        out_shape=[jax.ShapeDtypeStruct((D, D), _F32),
                   jax.ShapeDtypeStruct((D, D), _F32),
                   jax.ShapeDtypeStruct((1, D), _F32)] * 2,
    )(Wo0, Wn0, Wf0, bo0.reshape(1, D), bn0.reshape(1, D), bf0.reshape(1, D),
      Wo1, Wn1, Wf1, bo1.reshape(1, D), bn1.reshape(1, D), bf1.reshape(1, D))

    embs = []
    xin = x
    for (wco, wcn, bc) in ((wco0, wcn0, bc0), (wco1, wcn1, bc1)):
        ho, hn = _run_mm(xin, wco, wcn, dvo, dvn)
        So, Sn = _run_agg(ho.reshape(2 * N, HD), hn.reshape(2 * N, HD),
                          srco, dsto, ewo, srcn, dstn, ewn, z2d)
        xin = _run_comb(So.reshape(2, N, HD), Sn.reshape(2, N, HD),
                        ho, hn, dvo, dvn, bc)
        embs.append(_run_segmax(xin, ids2))
    return tuple(embs)
